# Initial kernel scaffold; baseline (speedup 1.0000x reference)
import jax
import jax.numpy as jnp
from jax.experimental import pallas as pl


def kernel(inputs, token_table, pos_table):
    raise NotImplementedError("write your pallas kernel here")


# speedup vs baseline: 11.2664x; 11.2664x over previous
import functools

import jax
import jax.numpy as jnp
from jax import lax
from jax.experimental import pallas as pl
from jax.experimental.pallas import tpu as pltpu
from jax.experimental.pallas import tpu_sc as plsc

D = 64
S = 65
V = 38
B = 16384
NC, NS, L = 2, 16, 16
NW = NC * NS
ROWS = B * S
ROWS_PER_W = ROWS // NW
GROUP = 16 * S
GROUPS = ROWS_PER_W // GROUP
FR = S * V
SPT = 5
GCH = 104
NG = GROUP // GCH


def _body(inputs_hbm, token_hbm, pos_hbm, out_hbm,
          token_v, pos_v, fused_s, spmem, offs, idx, rows_v, gsem):
    cid = lax.axis_index("c")
    sid = lax.axis_index("s")
    wid = sid * NC + cid
    tid = sid

    pltpu.sync_copy(token_hbm, token_v)
    pltpu.sync_copy(pos_hbm, pos_v)
    for k in range(SPT):
        s = lax.min(tid * SPT + k, S - 1)
        pos_row = [pos_v[pl.ds(s * D + j * L, L)] for j in range(D // L)]
        for v in range(V):
            for j in range(D // L):
                fused_s[v, pl.ds(j * L, L)] = (
                    token_v[pl.ds(v * D + j * L, L)] + pos_row[j])
        pltpu.sync_copy(fused_s, spmem.at[pl.ds(s * V, V)])
    plsc.subcore_barrier()

    for k in range(GROUP // L):
        p = lax.broadcasted_iota(jnp.int32, (L,), 0) + (k * L)
        offs[pl.ds(k * L, L)] = (p % S) * V

    row_base = wid * ROWS_PER_W

    def group_body(g, carry):
        start = row_base + g * GROUP
        pltpu.sync_copy(inputs_hbm.at[pl.ds(start, GROUP)], idx)
        for k in range(GROUP // L):
            sl = pl.ds(k * L, L)
            idx[sl] = idx[sl] + offs[sl]
        cps = [
            pltpu.async_copy(
                spmem.at[idx.at[pl.ds(j * GCH, GCH)]],
                rows_v.at[pl.ds(j * GCH, GCH)], gsem)
            for j in range(NG)
        ]
        for c in cps:
            c.wait()
        pltpu.sync_copy(rows_v, out_hbm.at[pl.ds(start, GROUP)])
        return carry

    lax.fori_loop(0, GROUPS, group_body, 0)


@jax.jit
def kernel(inputs, token_table, pos_table):
    mesh = plsc.VectorSubcoreMesh(
        core_axis_name="c", subcore_axis_name="s",
        num_cores=NC, num_subcores=NS)
    run = functools.partial(
        pl.kernel,
        out_type=jax.ShapeDtypeStruct((ROWS, D), jnp.float32),
        mesh=mesh,
        scratch_types=[
            pltpu.VMEM((V * D,), jnp.float32),
            pltpu.VMEM((S * D,), jnp.float32),
            pltpu.VMEM((V, D), jnp.float32),
            pltpu.VMEM_SHARED((FR, D), jnp.float32),
            pltpu.VMEM((GROUP,), jnp.int32),
            pltpu.VMEM((GROUP,), jnp.int32),
            pltpu.VMEM((GROUP, D), jnp.float32),
            pltpu.SemaphoreType.DMA,
        ],
        compiler_params=pltpu.CompilerParams(use_tc_tiling_on_sc=False),
    )(_body)
    out = run(inputs.reshape(ROWS), token_table.reshape(V * D),
              pos_table.reshape(S * D))
    return out.reshape(B, S, D)


# speedup vs baseline: 12.1509x; 1.0785x over previous
import functools

import jax
import jax.numpy as jnp
from jax import lax
from jax.experimental import pallas as pl
from jax.experimental.pallas import tpu as pltpu
from jax.experimental.pallas import tpu_sc as plsc

D = 64
S = 65
V = 38
B = 16384
NC, NS, L = 2, 16, 16
NW = NC * NS
ROWS = B * S
ROWS_PER_W = ROWS // NW
GROUP = 16 * S
GROUPS = ROWS_PER_W // GROUP
FR = S * V
SPT = 5
GCH = 104
HALF = GROUP // 2
NGH = HALF // GCH


def _body(inputs_hbm, token_hbm, pos_hbm, out_hbm,
          token_v, pos_v, fused_s, spmem, offs, idx,
          rows0, rows1, gsem, osem0, osem1):
    cid = lax.axis_index("c")
    sid = lax.axis_index("s")
    wid = sid * NC + cid
    tid = sid

    pltpu.sync_copy(token_hbm, token_v)
    pltpu.sync_copy(pos_hbm, pos_v)
    for k in range(SPT):
        s = lax.min(tid * SPT + k, S - 1)
        pos_row = [pos_v[pl.ds(s * D + j * L, L)] for j in range(D // L)]
        for v in range(V):
            for j in range(D // L):
                fused_s[v, pl.ds(j * L, L)] = (
                    token_v[pl.ds(v * D + j * L, L)] + pos_row[j])
        pltpu.sync_copy(fused_s, spmem.at[pl.ds(s * V, V)])
    plsc.subcore_barrier()

    for k in range(GROUP // L):
        p = lax.broadcasted_iota(jnp.int32, (L,), 0) + (k * L)
        offs[pl.ds(k * L, L)] = (p % S) * V

    row_base = wid * ROWS_PER_W
    bufs = ((rows0, osem0), (rows1, osem1))

    def emit_group(g, first):
        start = row_base + g * GROUP
        pltpu.sync_copy(inputs_hbm.at[pl.ds(start, GROUP)], idx)
        for k in range(GROUP // L):
            sl = pl.ds(k * L, L)
            idx[sl] = idx[sl] + offs[sl]
        for half, (buf, osem) in enumerate(bufs):
            if not first:
                pltpu.make_async_copy(
                    buf, out_hbm.at[pl.ds(0, HALF)], osem).wait()
            cps = [
                pltpu.async_copy(
                    spmem.at[idx.at[pl.ds(half * HALF + j * GCH, GCH)]],
                    buf.at[pl.ds(j * GCH, GCH)], gsem)
                for j in range(NGH)
            ]
            for c in cps:
                c.wait()
            pltpu.async_copy(
                buf, out_hbm.at[pl.ds(start + half * HALF, HALF)], osem)

    emit_group(0, True)
    lax.fori_loop(1, GROUPS, lambda g, c: (emit_group(g, False), c)[1], 0)
    for buf, osem in bufs:
        pltpu.make_async_copy(buf, out_hbm.at[pl.ds(0, HALF)], osem).wait()


@jax.jit
def kernel(inputs, token_table, pos_table):
    mesh = plsc.VectorSubcoreMesh(
        core_axis_name="c", subcore_axis_name="s",
        num_cores=NC, num_subcores=NS)
    run = functools.partial(
        pl.kernel,
        out_type=jax.ShapeDtypeStruct((ROWS, D), jnp.float32),
        mesh=mesh,
        scratch_types=[
            pltpu.VMEM((V * D,), jnp.float32),
            pltpu.VMEM((S * D,), jnp.float32),
            pltpu.VMEM((V, D), jnp.float32),
            pltpu.VMEM_SHARED((FR, D), jnp.float32),
            pltpu.VMEM((GROUP,), jnp.int32),
            pltpu.VMEM((GROUP,), jnp.int32),
            pltpu.VMEM((HALF, D), jnp.float32),
            pltpu.VMEM((HALF, D), jnp.float32),
            pltpu.SemaphoreType.DMA,
            pltpu.SemaphoreType.DMA,
            pltpu.SemaphoreType.DMA,
        ],
        compiler_params=pltpu.CompilerParams(use_tc_tiling_on_sc=False),
    )(_body)
    out = run(inputs.reshape(ROWS), token_table.reshape(V * D),
              pos_table.reshape(S * D))
    return out.reshape(B, S, D)
